# baseline (speedup 1.0000x reference)
import functools
import math

import jax
import jax.numpy as jnp
from jax import lax
from jax.experimental import pallas as pl
from jax.experimental.pallas import tpu as pltpu
from jax.experimental.pallas import tpu_sc as plsc

_info = plsc.get_sparse_core_info()
_NC, _NS, _NL = _info.num_cores, _info.num_subcores, _info.num_lanes
_NW = _NC * _NS


def _rtne16(u):
    return (u + jnp.uint32(0x7FFF) + ((u >> 16) & jnp.uint32(1))) >> 16


def _pack_body(t_ref, o_ref):
    x = t_ref[...]
    u = lax.bitcast_convert_type(x, jnp.uint32)
    lo = _rtne16(u[:32, :])
    hi = _rtne16(u[32:, :])
    o_ref[...] = lax.bitcast_convert_type((hi << 16) | lo, jnp.int32)


@functools.cache
def _make_pack(D, V, dtype):
    W = 2048
    grid = (V + W - 1) // W
    return pl.pallas_call(
        _pack_body,
        grid=(grid,),
        in_specs=[pl.BlockSpec((D, W), lambda i: (0, i))],
        out_specs=pl.BlockSpec((D // 2, W), lambda i: (0, i)),
        out_shape=jax.ShapeDtypeStruct((D // 2, V), jnp.int32),
    )


@functools.cache
def _make_lookup(BT, T, V, D, dtype):
    assert BT % 128 == 0 and D % 8 == 0
    n_bt = BT // 128
    n_blk = BT // _NL
    scale = dtype.type(math.sqrt(D))
    mesh = plsc.VectorSubcoreMesh(core_axis_name="c", subcore_axis_name="s")

    tch = 20
    assert T % tch == 0 and tch % 2 == 0
    n_tc = T // tch

    def body(tok_hbm, pak_hbm, out_hbm, row_v, tok_v, out_v, tok_s, *sems):
        tsem, osem = sems[:2], sems[2:]
        sid = lax.axis_index("s")
        wid = sid * _NC + lax.axis_index("c")
        col = BT // _NS
        dt0 = wid // 8
        ds0 = lax.rem(wid, 8)

        def tok_load(t, p):
            pltpu.async_copy(tok_s.at[t], tok_v.at[p], tsem[p])

        def tok_wait(t, p):
            pltpu.make_async_copy(tok_s.at[t], tok_v.at[p], tsem[p]).wait()

        def out_write(t, p, h):
            pltpu.async_copy(out_v.at[p, h],
                             out_hbm.at[t, dt0 + 4 * h, :, ds0, :],
                             osem[2 * p + h])

        def out_wait(t, p, h):
            pltpu.make_async_copy(out_v.at[p, h],
                                  out_hbm.at[t, dt0 + 4 * h, :, ds0, :],
                                  osem[2 * p + h]).wait()

        pltpu.sync_copy(pak_hbm.at[wid], row_v)

        def tc_body(tc, carry):
            plsc.subcore_barrier()
            pltpu.sync_copy(
                tok_hbm.at[pl.ds(tc * tch, tch), pl.ds(sid * col, col)],
                tok_s.at[:, pl.ds(sid * col, col)])
            plsc.subcore_barrier()
            tok_load(0, 0)
            tok_load(1, 1)

            def t_group(g, carry):
                for p in range(2):
                    tl = 2 * g + p
                    t = tc * tch + tl
                    tok_wait(tl, p)

                    first = jnp.logical_and(tc == 0, g == 0)

                    @pl.when(jnp.logical_not(first))
                    def _(t=t, p=p):
                        for h in range(2):
                            out_wait(t, p, h)

                    for k0 in range(0, n_blk, 8):
                        idxs = [tok_v[p, pl.ds((k0 + i) * _NL, _NL)]
                                for i in range(8)]
                        gs = [plsc.load_gather(row_v, [ix]) for ix in idxs]
                        los = [plsc.bitcast(lax.shift_left(g_, 16),
                                            dtype) * scale for g_ in gs]
                        his = [plsc.bitcast(
                            jnp.bitwise_and(g_, jnp.int32(-65536)),
                            dtype) * scale for g_ in gs]
                        for i in range(8):
                            k = k0 + i
                            sl = pl.ds((k % 8) * _NL, _NL)
                            out_v[p, 0, k // 8, sl] = los[i]
                            out_v[p, 1, k // 8, sl] = his[i]

                    @pl.when(g < tch // 2 - 1)
                    def _(tl=tl, p=p):
                        tok_load(tl + 2, p)

                    for h in range(2):
                        out_write(t, p, h)
                return carry

            lax.fori_loop(0, tch // 2, t_group, 0)
            return carry

        lax.fori_loop(0, n_tc, tc_body, 0)
        for p in range(2):
            for h in range(2):
                out_wait(T - 2 + p, p, h)

    return pl.kernel(
        body,
        out_type=jax.ShapeDtypeStruct((T, D // 8, n_bt, 8, 128), dtype),
        mesh=mesh,
        scratch_types=[
            pltpu.VMEM((V,), jnp.int32),
            pltpu.VMEM((2, BT), jnp.int32),
            pltpu.VMEM((2, 2, n_bt, 128), dtype),
            pltpu.VMEM_SHARED((tch, BT), jnp.int32),
        ] + [pltpu.SemaphoreType.DMA] * 6,
        compiler_params=pltpu.CompilerParams(
            use_tc_tiling_on_sc=False, needs_layout_passes=False),
    )


def kernel(tokens, table):
    BT, T = tokens.shape
    V, D = table.shape
    tokT = jnp.swapaxes(tokens, 0, 1).astype(jnp.int32)
    tabT = jnp.swapaxes(table, 0, 1)
    packed = _make_pack(D, V, jnp.dtype(table.dtype))(tabT)
    y5 = _make_lookup(BT, T, V, D, jnp.dtype(table.dtype))(tokT, packed)
    return y5.transpose(2, 4, 0, 1, 3).reshape(BT, T, D)


# speedup vs baseline: 1.0040x; 1.0040x over previous
import functools
import math

import jax
import jax.numpy as jnp
from jax import lax
from jax.experimental import pallas as pl
from jax.experimental.pallas import tpu as pltpu
from jax.experimental.pallas import tpu_sc as plsc

_info = plsc.get_sparse_core_info()
_NC, _NS, _NL = _info.num_cores, _info.num_subcores, _info.num_lanes
_NW = _NC * _NS


def _rtne16(u):
    return (u + jnp.uint32(0x7FFF) + ((u >> 16) & jnp.uint32(1))) >> 16


def _pack_body(scale, t_ref, o_ref):
    x = t_ref[...] * scale
    u = lax.bitcast_convert_type(x, jnp.uint32)
    lo = _rtne16(u[:32, :])
    hi = _rtne16(u[32:, :])
    o_ref[...] = lax.bitcast_convert_type((hi << 16) | lo, jnp.int32)


@functools.cache
def _make_pack(D, V, dtype):
    W = 2048
    grid = (V + W - 1) // W
    return pl.pallas_call(
        functools.partial(_pack_body, jnp.dtype(dtype).type(math.sqrt(D))),
        grid=(grid,),
        in_specs=[pl.BlockSpec((D, W), lambda i: (0, i))],
        out_specs=pl.BlockSpec((D // 2, W), lambda i: (0, i)),
        out_shape=jax.ShapeDtypeStruct((D // 2, V), jnp.int32),
    )


@functools.cache
def _make_lookup(BT, T, V, D, dtype):
    assert BT % 128 == 0 and D % 8 == 0
    n_bt = BT // 128
    n_blk = BT // _NL
    scale = dtype.type(math.sqrt(D))
    mesh = plsc.VectorSubcoreMesh(core_axis_name="c", subcore_axis_name="s")

    tch = 20
    assert T % tch == 0 and tch % 2 == 0
    n_tc = T // tch

    def body(tok_hbm, pak_hbm, out_hbm, row_v, tok_v, out_v, tok_s, *sems):
        tsem, osem = sems[:2], sems[2:]
        sid = lax.axis_index("s")
        wid = sid * _NC + lax.axis_index("c")
        col = BT // _NS
        dt0 = wid // 8
        ds0 = lax.rem(wid, 8)

        def tok_load(t, p):
            pltpu.async_copy(tok_s.at[t], tok_v.at[p], tsem[p])

        def tok_wait(t, p):
            pltpu.make_async_copy(tok_s.at[t], tok_v.at[p], tsem[p]).wait()

        def out_write(t, p, h):
            pltpu.async_copy(out_v.at[p, h],
                             out_hbm.at[t, dt0 + 4 * h, :, ds0, :],
                             osem[2 * p + h])

        def out_wait(t, p, h):
            pltpu.make_async_copy(out_v.at[p, h],
                                  out_hbm.at[t, dt0 + 4 * h, :, ds0, :],
                                  osem[2 * p + h]).wait()

        pltpu.sync_copy(pak_hbm.at[wid], row_v)

        def tc_body(tc, carry):
            plsc.subcore_barrier()
            pltpu.sync_copy(
                tok_hbm.at[pl.ds(tc * tch, tch), pl.ds(sid * col, col)],
                tok_s.at[:, pl.ds(sid * col, col)])
            plsc.subcore_barrier()
            tok_load(0, 0)
            tok_load(1, 1)

            def t_group(g, carry):
                for p in range(2):
                    tl = 2 * g + p
                    t = tc * tch + tl
                    tok_wait(tl, p)

                    first = jnp.logical_and(tc == 0, g == 0)

                    @pl.when(jnp.logical_not(first))
                    def _(t=t, p=p):
                        for h in range(2):
                            out_wait(t, p, h)

                    for k0 in range(0, n_blk, 8):
                        idxs = [tok_v[p, pl.ds((k0 + i) * _NL, _NL)]
                                for i in range(8)]
                        gs = [plsc.load_gather(row_v, [ix]) for ix in idxs]
                        los = [plsc.bitcast(lax.shift_left(g_, 16), dtype)
                               for g_ in gs]
                        his = [plsc.bitcast(
                            jnp.bitwise_and(g_, jnp.int32(-65536)), dtype)
                            for g_ in gs]
                        for i in range(8):
                            k = k0 + i
                            sl = pl.ds((k % 8) * _NL, _NL)
                            out_v[p, 0, k // 8, sl] = los[i]
                            out_v[p, 1, k // 8, sl] = his[i]

                    @pl.when(g < tch // 2 - 1)
                    def _(tl=tl, p=p):
                        tok_load(tl + 2, p)

                    for h in range(2):
                        out_write(t, p, h)
                return carry

            lax.fori_loop(0, tch // 2, t_group, 0)
            return carry

        lax.fori_loop(0, n_tc, tc_body, 0)
        for p in range(2):
            for h in range(2):
                out_wait(T - 2 + p, p, h)

    return pl.kernel(
        body,
        out_type=jax.ShapeDtypeStruct((T, D // 8, n_bt, 8, 128), dtype),
        mesh=mesh,
        scratch_types=[
            pltpu.VMEM((V,), jnp.int32),
            pltpu.VMEM((2, BT), jnp.int32),
            pltpu.VMEM((2, 2, n_bt, 128), dtype),
            pltpu.VMEM_SHARED((tch, BT), jnp.int32),
        ] + [pltpu.SemaphoreType.DMA] * 6,
        compiler_params=pltpu.CompilerParams(
            use_tc_tiling_on_sc=False, needs_layout_passes=False),
    )


def kernel(tokens, table):
    BT, T = tokens.shape
    V, D = table.shape
    tokT = jnp.swapaxes(tokens, 0, 1).astype(jnp.int32)
    tabT = jnp.swapaxes(table, 0, 1)
    packed = _make_pack(D, V, jnp.dtype(table.dtype))(tabT)
    y5 = _make_lookup(BT, T, V, D, jnp.dtype(table.dtype))(tokT, packed)
    return y5.transpose(2, 4, 0, 1, 3).reshape(BT, T, D)


# speedup vs baseline: 1.0675x; 1.0633x over previous
import functools
import math

import jax
import jax.numpy as jnp
from jax import lax
from jax.experimental import pallas as pl
from jax.experimental.pallas import tpu as pltpu
from jax.experimental.pallas import tpu_sc as plsc

_info = plsc.get_sparse_core_info()
_NC, _NS, _NL = _info.num_cores, _info.num_subcores, _info.num_lanes
_NW = _NC * _NS


def _rtne16(u):
    return (u + jnp.uint32(0x7FFF) + ((u >> 16) & jnp.uint32(1))) >> 16


def _pack_body(scale, t_ref, o_ref):
    x = t_ref[...] * scale
    u = lax.bitcast_convert_type(x, jnp.uint32)
    lo = _rtne16(u[:32, :])
    hi = _rtne16(u[32:, :])
    o_ref[...] = lax.bitcast_convert_type((hi << 16) | lo, jnp.int32)


@functools.cache
def _make_pack(D, V, dtype):
    W = 2048
    grid = (V + W - 1) // W
    return pl.pallas_call(
        functools.partial(_pack_body, jnp.dtype(dtype).type(math.sqrt(D))),
        grid=(grid,),
        in_specs=[pl.BlockSpec((D, W), lambda i: (0, i))],
        out_specs=pl.BlockSpec((D // 2, W), lambda i: (0, i)),
        out_shape=jax.ShapeDtypeStruct((D // 2, V), jnp.int32),
    )


@functools.cache
def _make_lookup(BT, T, V, D, dtype):
    assert BT % 128 == 0 and D % 8 == 0
    n_bt = BT // 128
    n_blk = BT // _NL
    scale = dtype.type(math.sqrt(D))
    mesh = plsc.VectorSubcoreMesh(core_axis_name="c", subcore_axis_name="s")

    tch = 10
    assert T % tch == 0 and tch % 2 == 0
    n_tc = T // tch

    def body(tok_hbm, pak_hbm, out_hbm, row_v, tok_v, out_v, tok_s, *sems):
        tsem, osem, ssem = sems[:2], sems[2:6], sems[6]
        sid = lax.axis_index("s")
        wid = sid * _NC + lax.axis_index("c")
        col = BT // _NS
        dt0 = wid // 8
        ds0 = lax.rem(wid, 8)

        def tok_load(q, t, p):
            pltpu.async_copy(tok_s.at[q, t], tok_v.at[p], tsem[p])

        def tok_wait(q, t, p):
            pltpu.make_async_copy(tok_s.at[q, t], tok_v.at[p], tsem[p]).wait()

        def stage(tc, q, sync):
            src = tok_hbm.at[pl.ds(tc * tch, tch), pl.ds(sid * col, col)]
            dst = tok_s.at[q, :, pl.ds(sid * col, col)]
            if sync:
                pltpu.sync_copy(src, dst)
            else:
                pltpu.async_copy(src, dst, ssem)

        def stage_wait(tc, q):
            pltpu.make_async_copy(
                tok_hbm.at[pl.ds(tc * tch, tch), pl.ds(sid * col, col)],
                tok_s.at[q, :, pl.ds(sid * col, col)], ssem).wait()

        def out_write(t, p, h):
            pltpu.async_copy(out_v.at[p, h],
                             out_hbm.at[t, dt0 + 4 * h, :, ds0, :],
                             osem[2 * p + h])

        def out_wait(t, p, h):
            pltpu.make_async_copy(out_v.at[p, h],
                                  out_hbm.at[t, dt0 + 4 * h, :, ds0, :],
                                  osem[2 * p + h]).wait()

        pltpu.sync_copy(pak_hbm.at[wid], row_v)

        def tc_body(tc, carry):
            q = lax.rem(tc, 2)

            @pl.when(tc + 1 < n_tc)
            def _():
                stage(tc + 1, 1 - q, sync=False)

            tok_load(q, 0, 0)
            tok_load(q, 1, 1)

            def t_group(g, carry):
                for p in range(2):
                    tl = 2 * g + p
                    t = tc * tch + tl
                    tok_wait(q, tl, p)

                    first = jnp.logical_and(tc == 0, g == 0)

                    @pl.when(jnp.logical_not(first))
                    def _(t=t, p=p):
                        for h in range(2):
                            out_wait(t, p, h)

                    for k0 in range(0, n_blk, 8):
                        idxs = [tok_v[p, pl.ds((k0 + i) * _NL, _NL)]
                                for i in range(8)]
                        gs = [plsc.load_gather(row_v, [ix]) for ix in idxs]
                        los = [plsc.bitcast(lax.shift_left(g_, 16), dtype)
                               for g_ in gs]
                        his = [plsc.bitcast(
                            jnp.bitwise_and(g_, jnp.int32(-65536)), dtype)
                            for g_ in gs]
                        for i in range(8):
                            k = k0 + i
                            sl = pl.ds((k % 8) * _NL, _NL)
                            out_v[p, 0, k // 8, sl] = los[i]
                            out_v[p, 1, k // 8, sl] = his[i]

                    @pl.when(g < tch // 2 - 1)
                    def _(tl=tl, p=p):
                        tok_load(q, tl + 2, p)

                    for h in range(2):
                        out_write(t, p, h)
                return carry

            lax.fori_loop(0, tch // 2, t_group, 0)

            @pl.when(tc + 1 < n_tc)
            def _():
                stage_wait(tc + 1, 1 - q)

            plsc.subcore_barrier()
            return carry

        stage(0, 0, sync=True)
        plsc.subcore_barrier()
        lax.fori_loop(0, n_tc, tc_body, 0)
        for p in range(2):
            for h in range(2):
                out_wait(T - 2 + p, p, h)

    return pl.kernel(
        body,
        out_type=jax.ShapeDtypeStruct((T, D // 8, n_bt, 8, 128), dtype),
        mesh=mesh,
        scratch_types=[
            pltpu.VMEM((V,), jnp.int32),
            pltpu.VMEM((2, BT), jnp.int32),
            pltpu.VMEM((2, 2, n_bt, 128), dtype),
            pltpu.VMEM_SHARED((2, tch, BT), jnp.int32),
        ] + [pltpu.SemaphoreType.DMA] * 7,
        compiler_params=pltpu.CompilerParams(
            use_tc_tiling_on_sc=False, needs_layout_passes=False),
    )


def kernel(tokens, table):
    BT, T = tokens.shape
    V, D = table.shape
    tokT = jnp.swapaxes(tokens, 0, 1).astype(jnp.int32)
    tabT = jnp.swapaxes(table, 0, 1)
    packed = _make_pack(D, V, jnp.dtype(table.dtype))(tabT)
    y5 = _make_lookup(BT, T, V, D, jnp.dtype(table.dtype))(tokT, packed)
    return y5.transpose(2, 4, 0, 1, 3).reshape(BT, T, D)


# speedup vs baseline: 1.2057x; 1.1294x over previous
import functools
import math

import jax
import jax.numpy as jnp
from jax import lax
from jax.experimental import pallas as pl
from jax.experimental.pallas import tpu as pltpu
from jax.experimental.pallas import tpu_sc as plsc

_info = plsc.get_sparse_core_info()
_NC, _NS, _NL = _info.num_cores, _info.num_subcores, _info.num_lanes
_NW = _NC * _NS


@functools.cache
def _make_lookup(BT, T, V, D, dtype):
    assert BT % 128 == 0 and D % 8 == 0
    n_bt = BT // 128
    d_per_w = D // _NW
    n_blk = BT // _NL
    scale = dtype.type(math.sqrt(D))
    mesh = plsc.VectorSubcoreMesh(core_axis_name="c", subcore_axis_name="s")

    tch = 20
    assert T % tch == 0 and tch % 2 == 0
    n_tc = T // tch
    n_vis = d_per_w * n_tc

    def body(tok_hbm, tab_hbm, out_hbm, row_v, tok_v, out_v, tok_s, *sems):
        tsem, osem, ssem = sems[:2], sems[2:4], sems[4]
        sid = lax.axis_index("s")
        wid = sid * _NC + lax.axis_index("c")
        col = BT // _NS

        def tok_load(q, t, p):
            pltpu.async_copy(tok_s.at[q, t], tok_v.at[p], tsem[p])

        def tok_wait(q, t, p):
            pltpu.make_async_copy(tok_s.at[q, t], tok_v.at[p], tsem[p]).wait()

        def stage(tc, q, sync):
            src = tok_hbm.at[pl.ds(tc * tch, tch), pl.ds(sid * col, col)]
            dst = tok_s.at[q, :, pl.ds(sid * col, col)]
            if sync:
                pltpu.sync_copy(src, dst)
            else:
                pltpu.async_copy(src, dst, ssem)

        def stage_wait(tc, q):
            pltpu.make_async_copy(
                tok_hbm.at[pl.ds(tc * tch, tch), pl.ds(sid * col, col)],
                tok_s.at[q, :, pl.ds(sid * col, col)], ssem).wait()

        def out_write(t, p, dt, ds):
            pltpu.async_copy(out_v.at[p], out_hbm.at[t, dt, :, ds, :], osem[p])

        def out_wait(t, p, dt, ds):
            pltpu.make_async_copy(
                out_v.at[p], out_hbm.at[t, dt, :, ds, :], osem[p]).wait()

        stage(0, 0, sync=True)
        plsc.subcore_barrier()

        def vc_body(vc, carry):
            di = vc // n_tc
            tc = lax.rem(vc, n_tc)
            q = lax.rem(vc, 2)
            d = wid + di * _NW
            dt = d // 8
            ds = lax.rem(d, 8)

            @pl.when(tc == 0)
            def _():
                pltpu.sync_copy(tab_hbm.at[d], row_v)

            @pl.when(vc + 1 < n_vis)
            def _():
                stage(lax.rem(vc + 1, n_tc), 1 - q, sync=False)

            tok_load(q, 0, 0)
            tok_load(q, 1, 1)

            def t_group(g, carry):
                for p in range(2):
                    tl = 2 * g + p
                    t = tc * tch + tl
                    tok_wait(q, tl, p)

                    first = jnp.logical_and(vc == 0, g == 0)

                    @pl.when(jnp.logical_not(first))
                    def _(t=t, p=p):
                        out_wait(t, p, dt, ds)

                    for k0 in range(0, n_blk, 8):
                        idxs = [tok_v[p, pl.ds((k0 + i) * _NL, _NL)]
                                for i in range(8)]
                        vals = [plsc.load_gather(row_v, [ix]) * scale
                                for ix in idxs]
                        for i in range(8):
                            k = k0 + i
                            out_v[p, k // 8,
                                  pl.ds((k % 8) * _NL, _NL)] = vals[i]

                    @pl.when(g < tch // 2 - 1)
                    def _(tl=tl, p=p):
                        tok_load(q, tl + 2, p)

                    out_write(t, p, dt, ds)
                return carry

            lax.fori_loop(0, tch // 2, t_group, 0)

            @pl.when(vc + 1 < n_vis)
            def _():
                stage_wait(lax.rem(vc + 1, n_tc), 1 - q)

            plsc.subcore_barrier()
            return carry

        lax.fori_loop(0, n_vis, vc_body, 0)

        d_last = wid + (d_per_w - 1) * _NW
        for p in range(2):
            out_wait(T - 2 + p, p, d_last // 8, lax.rem(d_last, 8))

    return pl.kernel(
        body,
        out_type=jax.ShapeDtypeStruct((T, D // 8, n_bt, 8, 128), dtype),
        mesh=mesh,
        scratch_types=[
            pltpu.VMEM((V,), dtype),
            pltpu.VMEM((2, BT), jnp.int32),
            pltpu.VMEM((2, n_bt, 128), dtype),
            pltpu.VMEM_SHARED((2, tch, BT), jnp.int32),
        ] + [pltpu.SemaphoreType.DMA] * 5,
        compiler_params=pltpu.CompilerParams(
            use_tc_tiling_on_sc=False, needs_layout_passes=False),
    )


def kernel(tokens, table):
    BT, T = tokens.shape
    V, D = table.shape
    tokT = jnp.swapaxes(tokens, 0, 1).astype(jnp.int32)
    tabT = jnp.swapaxes(table, 0, 1)
    y5 = _make_lookup(BT, T, V, D, jnp.dtype(table.dtype))(tokT, tabT)
    return y5.transpose(2, 4, 0, 1, 3).reshape(BT, T, D)
